# baseline (speedup 1.0000x reference)
import jax
import jax.numpy as jnp
from jax.experimental import pallas as pl
from jax.experimental.pallas import tpu as pltpu
from jax.experimental.pallas import tpu_sc as plsc

_GATHER_WINDOW = 128
_VOCAB_TILE = 1280


def _sc_gather_rows(table, idx):
    n = idx.shape[1]
    d = table.shape[1]
    mesh = plsc.VectorSubcoreMesh(core_axis_name="c", subcore_axis_name="s")

    @pl.kernel(
        out_type=jax.ShapeDtypeStruct((n, d), table.dtype),
        mesh=mesh,
    )
    def gather_kernel(tab_hbm, idx_hbm, out_hbm):
        def body(idx_vmem, out_vmem):
            pltpu.sync_copy(tab_hbm.at[idx_vmem.at[0]], out_vmem)

        pltpu.emit_pipeline(
            body,
            grid=(n // _GATHER_WINDOW,),
            in_specs=[pl.BlockSpec((1, _GATHER_WINDOW), lambda i: (0, i))],
            out_specs=[pl.BlockSpec((_GATHER_WINDOW, d), lambda i: (i, 0))],
            core_axis_name=("c", "s"),
            dimension_semantics=(pltpu.PARALLEL,),
        )(idx_hbm, out_hbm)

    return gather_kernel(table, idx)


def _tc_decode(x, w, bias2d):
    m, k = x.shape
    v = w.shape[0]

    def mm(x_ref, w_ref, b_ref, o_ref):
        o_ref[...] = (
            jax.lax.dot_general(
                x_ref[...],
                w_ref[...],
                (((1,), (1,)), ((), ())),
                preferred_element_type=jnp.float32,
            )
            + b_ref[...]
        )

    return pl.pallas_call(
        mm,
        grid=(v // _VOCAB_TILE,),
        in_specs=[
            pl.BlockSpec((m, k), lambda j: (0, 0)),
            pl.BlockSpec((_VOCAB_TILE, k), lambda j: (j, 0)),
            pl.BlockSpec((1, _VOCAB_TILE), lambda j: (0, j)),
        ],
        out_specs=pl.BlockSpec((m, _VOCAB_TILE), lambda j: (0, j)),
        out_shape=jax.ShapeDtypeStruct((m, v), jnp.float32),
    )(x, w, bias2d)


def kernel(src, emb, bias):
    seq, batch = src.shape
    vocab, dim = emb.shape
    idx = src.reshape(1, seq * batch).astype(jnp.int32)
    emb_bf = emb.astype(jnp.bfloat16)
    emb_words = jax.lax.bitcast_convert_type(
        emb_bf.reshape(vocab, dim // 2, 2), jnp.int32)
    x_words = _sc_gather_rows(emb_words, idx)
    x = jax.lax.bitcast_convert_type(x_words, jnp.bfloat16).reshape(
        seq * batch, dim)
    out = _tc_decode(x, emb_bf, bias.reshape(1, vocab))
    return out.reshape(seq, batch, vocab)


# speedup vs baseline: 2.2184x; 2.2184x over previous
import jax
import jax.numpy as jnp
from jax.experimental import pallas as pl
from jax.experimental.pallas import tpu as pltpu
from jax.experimental.pallas import tpu_sc as plsc

_GATHER_WINDOW = 128
_VOCAB_TILE = 1280


def _sc_gather_rows(table, idx):
    n = idx.shape[1]
    d = table.shape[1]
    mesh = plsc.VectorSubcoreMesh(core_axis_name="c", subcore_axis_name="s")

    @pl.kernel(
        out_type=jax.ShapeDtypeStruct((n, d), table.dtype),
        mesh=mesh,
    )
    def gather_kernel(tab_hbm, idx_hbm, out_hbm):
        def body(idx_vmem, out_vmem):
            pltpu.sync_copy(tab_hbm.at[idx_vmem.at[0]], out_vmem)

        pltpu.emit_pipeline(
            body,
            grid=(n // _GATHER_WINDOW,),
            in_specs=[pl.BlockSpec((1, _GATHER_WINDOW), lambda i: (0, i))],
            out_specs=[pl.BlockSpec((_GATHER_WINDOW, d), lambda i: (i, 0))],
            core_axis_name=("c", "s"),
            dimension_semantics=(pltpu.PARALLEL,),
        )(idx_hbm, out_hbm)

    return gather_kernel(table, idx)


def _tc_decode(x_bf, w, bias2d):
    m, k = x_bf.shape
    v = w.shape[0]

    def mm(x_ref, w_ref, b_ref, o_ref):
        o_ref[...] = (
            jax.lax.dot_general(
                x_ref[...],
                w_ref[...].astype(jnp.bfloat16),
                (((1,), (1,)), ((), ())),
                preferred_element_type=jnp.float32,
            )
            + b_ref[...]
        )

    return pl.pallas_call(
        mm,
        grid=(v // _VOCAB_TILE,),
        in_specs=[
            pl.BlockSpec((m, k), lambda j: (0, 0)),
            pl.BlockSpec((_VOCAB_TILE, k), lambda j: (j, 0)),
            pl.BlockSpec((1, _VOCAB_TILE), lambda j: (0, j)),
        ],
        out_specs=pl.BlockSpec((m, _VOCAB_TILE), lambda j: (0, j)),
        out_shape=jax.ShapeDtypeStruct((m, v), jnp.float32),
    )(x_bf, w, bias2d)


def kernel(src, emb, bias):
    seq, batch = src.shape
    vocab, dim = emb.shape
    n = seq * batch
    idx = src.reshape(n, 1).astype(jnp.int32)
    idx2 = (idx * 2 + jnp.arange(2, dtype=jnp.int32)).reshape(1, 2 * n)
    halves = _sc_gather_rows(emb.reshape(2 * vocab, dim // 2), idx2)
    x_bf = halves.reshape(n, dim).astype(jnp.bfloat16)
    out = _tc_decode(x_bf, emb, bias.reshape(1, vocab))
    return out.reshape(seq, batch, vocab)


# speedup vs baseline: 2.6777x; 1.2070x over previous
import jax
import jax.numpy as jnp
from jax import lax
from jax.experimental import pallas as pl
from jax.experimental.pallas import tpu as pltpu
from jax.experimental.pallas import tpu_sc as plsc

_NUM_CORES = 2
_NUM_SUBCORES = 16
_NUM_WORKERS = _NUM_CORES * _NUM_SUBCORES

_VOCAB_TILE = 1280


def _sc_gather_rows(table, idx):
    n = idx.shape[0]
    d = table.shape[1]
    b_per_w = n // _NUM_WORKERS
    mesh = plsc.VectorSubcoreMesh(core_axis_name="c", subcore_axis_name="s")

    @pl.kernel(
        out_type=jax.ShapeDtypeStruct((n, d), table.dtype),
        mesh=mesh,
        scratch_types=[
            pltpu.VMEM((b_per_w,), jnp.int32),
            pltpu.VMEM((b_per_w, d), table.dtype),
            pltpu.SemaphoreType.DMA,
        ],
    )
    def gather_kernel(tab_hbm, idx_hbm, out_hbm, idx_v, rows_v, sem):
        wid = lax.axis_index("s") * _NUM_CORES + lax.axis_index("c")
        base = wid * b_per_w
        pltpu.sync_copy(idx_hbm.at[pl.ds(base, b_per_w)], idx_v)
        pltpu.async_copy(tab_hbm.at[idx_v], rows_v, sem).wait()
        pltpu.sync_copy(rows_v, out_hbm.at[pl.ds(base, b_per_w)])

    return gather_kernel(table, idx)


def _tc_decode(x_bf, w, bias2d):
    m, k = x_bf.shape
    v = w.shape[0]

    def mm(x_ref, w_ref, b_ref, o_ref):
        o_ref[...] = (
            jax.lax.dot_general(
                x_ref[...],
                w_ref[...].astype(jnp.bfloat16),
                (((1,), (1,)), ((), ())),
                preferred_element_type=jnp.float32,
            )
            + b_ref[...]
        )

    return pl.pallas_call(
        mm,
        grid=(v // _VOCAB_TILE,),
        in_specs=[
            pl.BlockSpec((m, k), lambda j: (0, 0)),
            pl.BlockSpec((_VOCAB_TILE, k), lambda j: (j, 0)),
            pl.BlockSpec((1, _VOCAB_TILE), lambda j: (0, j)),
        ],
        out_specs=pl.BlockSpec((m, _VOCAB_TILE), lambda j: (0, j)),
        out_shape=jax.ShapeDtypeStruct((m, v), jnp.float32),
    )(x_bf, w, bias2d)


def kernel(src, emb, bias):
    seq, batch = src.shape
    vocab, dim = emb.shape
    n = seq * batch
    idx = src.reshape(n).astype(jnp.int32)
    x = _sc_gather_rows(emb, idx)
    x_bf = x.astype(jnp.bfloat16)
    out = _tc_decode(x_bf, emb, bias.reshape(1, vocab))
    return out.reshape(seq, batch, vocab)


# speedup vs baseline: 5.2302x; 1.9533x over previous
import jax
import jax.numpy as jnp
from jax import lax
from jax.experimental import pallas as pl
from jax.experimental.pallas import tpu as pltpu
from jax.experimental.pallas import tpu_sc as plsc

_NUM_CORES = 2
_NUM_SUBCORES = 16
_NUM_WORKERS = _NUM_CORES * _NUM_SUBCORES

_VOCAB_TILE = 640


def _sc_gather_rows(table, idx):
    n = idx.shape[0]
    d = table.shape[1]
    b_per_w = n // _NUM_WORKERS
    mesh = plsc.VectorSubcoreMesh(core_axis_name="c", subcore_axis_name="s")

    @pl.kernel(
        out_type=jax.ShapeDtypeStruct((n, d), table.dtype),
        mesh=mesh,
        scratch_types=[
            pltpu.VMEM((b_per_w,), jnp.int32),
            pltpu.VMEM((b_per_w, d), table.dtype),
            pltpu.SemaphoreType.DMA,
        ],
    )
    def gather_kernel(tab_hbm, idx_hbm, out_hbm, idx_v, rows_v, sem):
        wid = lax.axis_index("s") * _NUM_CORES + lax.axis_index("c")
        base = wid * b_per_w
        pltpu.sync_copy(idx_hbm.at[pl.ds(base, b_per_w)], idx_v)
        pltpu.async_copy(tab_hbm.at[idx_v], rows_v, sem).wait()
        pltpu.sync_copy(rows_v, out_hbm.at[pl.ds(base, b_per_w)])

    return gather_kernel(table, idx)


def _tc_decode(x_bf, w, bias2d, seq, batch):
    k = x_bf.shape[1]
    v = w.shape[0]

    def mm(x_ref, w_ref, b_ref, o_ref):
        wb = w_ref[...].astype(jnp.bfloat16)
        for i in range(batch):
            o_ref[:, i, :] = (
                jax.lax.dot_general(
                    x_ref[i * seq:(i + 1) * seq, :],
                    wb,
                    (((1,), (1,)), ((), ())),
                    preferred_element_type=jnp.float32,
                )
                + b_ref[...]
            )

    return pl.pallas_call(
        mm,
        grid=(v // _VOCAB_TILE,),
        in_specs=[
            pl.BlockSpec((seq * batch, k), lambda j: (0, 0)),
            pl.BlockSpec((_VOCAB_TILE, k), lambda j: (j, 0)),
            pl.BlockSpec((1, _VOCAB_TILE), lambda j: (0, j)),
        ],
        out_specs=pl.BlockSpec((seq, batch, _VOCAB_TILE), lambda j: (0, 0, j)),
        out_shape=jax.ShapeDtypeStruct((seq, batch, v), jnp.float32),
    )(x_bf, w, bias2d)


def kernel(src, emb, bias):
    seq, batch = src.shape
    vocab, dim = emb.shape
    n = seq * batch
    idx = src.T.reshape(n).astype(jnp.int32)
    x = _sc_gather_rows(emb, idx)
    x_bf = x.astype(jnp.bfloat16)
    return _tc_decode(x_bf, emb, bias.reshape(1, vocab), seq, batch)
